# baseline (speedup 1.0000x reference)
import functools

import jax
import jax.numpy as jnp
from jax import lax
from jax.experimental import pallas as pl
from jax.experimental.pallas import tpu as pltpu
from jax.experimental.pallas import tpu_sc as plsc

NSEG = 512
DMID = 256
RA = 1280
NC, NS = 2, 16
NW = NC * NS
SEG_PER_W = NSEG // NW
CHUNK = 128
BUF = CHUNK + 8
LANE = 16

_LOG2E = 1.4426950408889634
_LN2 = 0.6931471805599453


def _softplus(x):
    t = jnp.exp2(-jnp.abs(x) * _LOG2E)
    return jnp.maximum(x, 0.0) + _LN2 * jnp.log2(1.0 + t)


def _count_body(ids_ref, ptr_ref):
    ids = ids_ref[...]
    lane = lax.broadcasted_iota(jnp.int32, (1, 128), 1)
    acc = jnp.zeros((1, 128), jnp.float32)
    for t in range(NW + 1):
        cnt = jnp.sum((ids < 16 * t).astype(jnp.float32))
        acc = acc + jnp.where(lane == t, cnt, 0.0)
    ptr_ref[...] = acc.astype(jnp.int32)


def _mlp_body(h_ref, w1_ref, b1_ref, y_ref):
    y_ref[...] = _softplus(
        jnp.dot(h_ref[...], w1_ref[...], preferred_element_type=jnp.float32)
        + b1_ref[...])


def _extract(vec, i):
    sel = jnp.where(lax.iota(jnp.int32, LANE) == i,
                    vec.astype(jnp.float32), 0.0)
    return jnp.sum(sel).astype(jnp.int32)


def _zeros16():
    return tuple(jnp.zeros((LANE,), jnp.float32) for _ in range(DMID // LANE))


def _segsum_body(y_hbm, b_hbm, ptr_hbm, out_hbm, ptr_v, idx_v, rows_v, acc_v,
                 *, n):
    c = lax.axis_index("c")
    s = lax.axis_index("s")
    w = c * NS + s
    seg0 = w * SEG_PER_W

    pltpu.sync_copy(ptr_hbm, ptr_v)
    pb = (w // 8) * 8
    pvec = ptr_v[pl.ds(pb, LANE)]
    lo = _extract(pvec, w - pb)
    hi = _extract(pvec, w - pb + 1)

    def zrow(i, _):
        def zcol(j, _):
            acc_v[i, pl.ds(j * LANE, LANE)] = jnp.zeros((LANE,), jnp.float32)
            return 0
        lax.fori_loop(0, DMID // LANE, zcol, 0)
        return 0
    lax.fori_loop(0, SEG_PER_W, zrow, 0)

    nch = lax.div(hi - lo + CHUNK - 1, CHUNK)

    def chunk_body(k, carry):
        cur, acc = carry
        base = lo + k * CHUNK
        base_c = jnp.minimum((base // 8) * 8, n - BUF)
        pltpu.sync_copy(y_hbm.at[pl.ds(base_c, BUF)], rows_v)
        pltpu.sync_copy(b_hbm.at[pl.ds(base_c, BUF)], idx_v)
        r0 = base - base_c
        r1 = r0 + jnp.minimum(hi - base, CHUNK)

        def row_body(r, rc):
            cur, acc = rc
            g = (r // LANE) * LANE
            seg = _extract(idx_v[pl.ds(g, LANE)], r - g)
            same = seg == cur

            @pl.when(jnp.logical_and(cur >= 0, jnp.logical_not(same)))
            def _flush():
                row = cur - seg0
                for j in range(DMID // LANE):
                    acc_v[row, pl.ds(j * LANE, LANE)] = acc[j]

            row = tuple(rows_v[r, pl.ds(j * LANE, LANE)]
                        for j in range(DMID // LANE))
            nacc = tuple(
                jnp.where(same, acc[j] + row[j], row[j])
                for j in range(DMID // LANE))
            return seg, nacc

        return lax.fori_loop(r0, r1, row_body, (cur, acc))

    cur, acc = lax.fori_loop(0, nch, chunk_body, (jnp.int32(-1), _zeros16()))

    @pl.when(cur >= 0)
    def _final_flush():
        row = cur - seg0
        for j in range(DMID // LANE):
            acc_v[row, pl.ds(j * LANE, LANE)] = acc[j]

    pltpu.sync_copy(acc_v, out_hbm.at[pl.ds(seg0, SEG_PER_W)])


def _tail_body(p_ref, w2_ref, b2_ref, w3_ref, b3_ref, out_ref):
    z = _softplus(p_ref[...])
    z = _softplus(
        jnp.dot(z, w2_ref[...], preferred_element_type=jnp.float32)
        + b2_ref[...])
    out_ref[...] = (
        jnp.dot(z, w3_ref[...], preferred_element_type=jnp.float32)
        + b3_ref[...])


def kernel(h, batch, W1, b1, W2, b2, W3, b3):
    n, dk = h.shape
    nblocks = n // RA
    batch32 = batch.astype(jnp.int32)

    ids_pad = jnp.pad(batch32, (0, (-n) % 1024), constant_values=2 ** 20)
    ids2d = ids_pad.reshape(-1, 128)
    ptr33 = pl.pallas_call(
        _count_body,
        grid=(1,),
        in_specs=[pl.BlockSpec(ids2d.shape, lambda i: (0, 0))],
        out_specs=pl.BlockSpec((1, 128), lambda i: (0, 0)),
        out_shape=jax.ShapeDtypeStruct((1, 128), jnp.int32),
    )(ids2d).reshape(128)

    y = pl.pallas_call(
        _mlp_body,
        grid=(nblocks,),
        in_specs=[
            pl.BlockSpec((RA, dk), lambda i: (i, 0)),
            pl.BlockSpec(W1.shape, lambda i: (0, 0)),
            pl.BlockSpec((1, DMID), lambda i: (0, 0)),
        ],
        out_specs=pl.BlockSpec((RA, DMID), lambda i: (i, 0)),
        out_shape=jax.ShapeDtypeStruct((n, DMID), jnp.float32),
    )(h, W1, b1.reshape(1, DMID))

    segsum = functools.partial(
        pl.kernel,
        mesh=plsc.VectorSubcoreMesh(core_axis_name="c", subcore_axis_name="s"),
        out_type=jax.ShapeDtypeStruct((NSEG, DMID), jnp.float32),
        scratch_types=[
            pltpu.VMEM((128,), jnp.int32),
            pltpu.VMEM((BUF,), jnp.int32),
            pltpu.VMEM((BUF, DMID), jnp.float32),
            pltpu.VMEM((SEG_PER_W, DMID), jnp.float32),
        ],
        compiler_params=pltpu.CompilerParams(needs_layout_passes=False),
    )(functools.partial(_segsum_body, n=n))
    seg = segsum(y, batch32, ptr33)

    out = pl.pallas_call(
        _tail_body,
        grid=(1,),
        in_specs=[
            pl.BlockSpec((NSEG, DMID), lambda i: (0, 0)),
            pl.BlockSpec(W2.shape, lambda i: (0, 0)),
            pl.BlockSpec((1, 64), lambda i: (0, 0)),
            pl.BlockSpec(W3.shape, lambda i: (0, 0)),
            pl.BlockSpec((1, 1), lambda i: (0, 0)),
        ],
        out_specs=pl.BlockSpec((NSEG, 1), lambda i: (0, 0)),
        out_shape=jax.ShapeDtypeStruct((NSEG, 1), jnp.float32),
    )(seg, W2, b2.reshape(1, 64), W3, b3.reshape(1, 1))
    return out
